# baseline (speedup 1.0000x reference)
import jax
import jax.numpy as jnp
from jax import lax
from jax.experimental import pallas as pl
from jax.experimental.pallas import tpu as pltpu
import jax.experimental.pallas.tpu_sc as plsc

N = 10000
E = 320000
D = 128

NC = 2
NS = 16
NW = NC * NS

CH = 16
GRP = 4
CHB = CH * GRP
NCHW = -(-E // (NW * CHB))
EPW = NCHW * CHB
EPAD = NW * EPW
NBUF = 4
JUNK = N
NA = N + 8
RPT = 624
ZR = 208
ZB = 16
TAIL = N - NS * RPT


def _sc_segment_sum_body(tab_hbm, packed_hbm, out_hbm, *scr):
    c = lax.axis_index("c")
    s = lax.axis_index("s")
    wid = c * NS + s
    packed_v = scr[0]
    srng = scr[1:1 + NBUF]
    drng = scr[1 + NBUF:1 + 2 * NBUF]
    rows = scr[1 + 2 * NBUF:1 + 3 * NBUF]
    zbuf = scr[1 + 3 * NBUF]
    acc = scr[2 + 3 * NBUF]
    gsem = scr[3 + 3 * NBUF:3 + 4 * NBUF]

    pltpu.sync_copy(packed_hbm.at[pl.ds(wid * EPW, EPW)], packed_v)

    def _unpack(j, b):
        for k in range(CHB // 16):
            p = packed_v[pl.ds(j * CHB + k * 16, 16)]
            srng[b][pl.ds(k * 16, 16)] = p & 0xFFFF
            drng[b][pl.ds(k * 16, 16)] = lax.shift_right_logical(p, 16)

    def _fire(b):
        for q in range(GRP):
            pltpu.async_copy(tab_hbm.at[srng[b].at[pl.ds(q * CH, CH)]],
                             rows[b].at[pl.ds(q * CH, CH), :], gsem[b])

    def _wait(b):
        pltpu.make_async_copy(tab_hbm.at[srng[b]], rows[b], gsem[b]).wait()

    for b in range(NBUF):
        _unpack(b, b)
        _fire(b)

    zero16 = jnp.zeros((16,), jnp.float32)

    def _zrow(r, _):
        def _zcol(k, _):
            zbuf[r, pl.ds(k * 16, 16)] = zero16
            return 0
        return lax.fori_loop(0, D // 16, _zcol, 0)

    lax.fori_loop(0, ZB, _zrow, 0)

    for j in range(RPT // ZB):
        pltpu.sync_copy(zbuf, acc.at[pl.ds(s * RPT + j * ZB, ZB), :])

    @pl.when(s == 0)
    def _zero_tail():
        pltpu.sync_copy(zbuf.at[pl.ds(0, TAIL), :],
                        acc.at[pl.ds(NS * RPT, TAIL), :])

    plsc.subcore_barrier()

    def _group(t, _):
        j0 = t * NBUF
        for b in range(NBUF):
            j = j0 + b
            _wait(b)
            pltpu.sync_copy(rows[b], acc.at[drng[b]], add=True)
            nxt = j + NBUF

            @pl.when(nxt < NCHW)
            def _refire():
                _unpack(nxt, b)
                _fire(b)
        return 0

    lax.fori_loop(0, NCHW // NBUF, _group, 0)
    for j in range((NCHW // NBUF) * NBUF, NCHW):
        b = j % NBUF
        _wait(b)
        pltpu.sync_copy(rows[b], acc.at[drng[b]], add=True)
    plsc.subcore_barrier()

    for j in range(RPT // ZR):
        r0 = s * RPT + j * ZR
        pltpu.sync_copy(acc.at[pl.ds(r0, ZR), :], out_hbm.at[c, pl.ds(r0, ZR), :])

    @pl.when(s == 0)
    def _write_tail():
        pltpu.sync_copy(acc.at[pl.ds(NS * RPT, TAIL), :],
                        out_hbm.at[c, pl.ds(NS * RPT, TAIL), :])


def _sc_segment_sum(table, packed):
    mesh = plsc.VectorSubcoreMesh(core_axis_name="c", subcore_axis_name="s")
    f = pl.kernel(
        _sc_segment_sum_body,
        out_type=jax.ShapeDtypeStruct((NC, N, D), jnp.float32),
        mesh=mesh,
        scratch_types=(
            [pltpu.VMEM((EPW,), jnp.int32)]
            + [pltpu.VMEM((CHB,), jnp.int32)] * (2 * NBUF)
            + [pltpu.VMEM((CHB, D), jnp.float32)] * NBUF
            + [pltpu.VMEM((ZB, D), jnp.float32),
               pltpu.VMEM_SHARED((NA, D), jnp.float32)]
            + [pltpu.SemaphoreType.DMA] * NBUF
        ),
    )
    return f(table, packed)


def _tc_relu_body(p_ref, w_ref, b_ref, o_ref):
    a = p_ref[0] + p_ref[1]
    h = jnp.dot(a, w_ref[...], preferred_element_type=jnp.float32)
    o_ref[...] = jnp.maximum(h + b_ref[...], 0.0)


def _tc_sigmoid_body(p_ref, w_ref, b_ref, o_ref):
    a = p_ref[0] + p_ref[1]
    h = jnp.dot(a, w_ref[...], preferred_element_type=jnp.float32)
    o_ref[...] = jax.nn.sigmoid(h + b_ref[...])


def _tc_layer(partials, w, b, body):
    rb = 2000
    grid = (N // rb,)
    return pl.pallas_call(
        body,
        grid=grid,
        in_specs=[
            pl.BlockSpec((NC, rb, D), lambda i: (0, i, 0)),
            pl.BlockSpec((D, D), lambda i: (0, 0)),
            pl.BlockSpec((1, D), lambda i: (0, 0)),
        ],
        out_specs=pl.BlockSpec((rb, D), lambda i: (i, 0)),
        out_shape=jax.ShapeDtypeStruct((N, D), jnp.float32),
    )(partials, w, b.reshape(1, D))


def kernel(x, edge_index, W1, b1, W2, b2):
    pad = EPAD - E
    src = jnp.concatenate([edge_index[0], jnp.zeros((pad,), jnp.int32)])
    dst = jnp.concatenate([edge_index[1], jnp.full((pad,), JUNK, jnp.int32)])
    packed = src + dst * 65536
    p1 = _sc_segment_sum(x, packed)
    h = _tc_layer(p1, W1, b1, _tc_relu_body)
    p2 = _sc_segment_sum(h, packed)
    return _tc_layer(p2, W2, b2, _tc_sigmoid_body)


# speedup vs baseline: 1.4191x; 1.4191x over previous
import jax
import jax.numpy as jnp
from jax import lax
from jax.experimental import pallas as pl
from jax.experimental.pallas import tpu as pltpu
import jax.experimental.pallas.tpu_sc as plsc

N = 10000
E = 320000
D = 128

NC = 2
NS = 16
NW = NC * NS

CH = 16
GRP = 1
CHB = CH * GRP
NCHW = -(-E // (NW * CHB))
EPW = NCHW * CHB
EPAD = NW * EPW
NBUF = 16
JUNK = N
NA = N + 8
RPT = 624
ZR = 208
ZB = 16
TAIL = N - NS * RPT


def _sc_segment_sum_body(tab_hbm, packed_hbm, out_hbm, *scr):
    c = lax.axis_index("c")
    s = lax.axis_index("s")
    wid = c * NS + s
    packed_v = scr[0]
    srng = scr[1:1 + NBUF]
    drng = scr[1 + NBUF:1 + 2 * NBUF]
    rows = scr[1 + 2 * NBUF:1 + 3 * NBUF]
    zbuf = scr[1 + 3 * NBUF]
    acc = scr[2 + 3 * NBUF]
    gsem = scr[3 + 3 * NBUF:3 + 4 * NBUF]

    pltpu.sync_copy(packed_hbm.at[pl.ds(wid * EPW, EPW)], packed_v)

    def _unpack(j, b):
        for k in range(CHB // 16):
            p = packed_v[pl.ds(j * CHB + k * 16, 16)]
            srng[b][pl.ds(k * 16, 16)] = p & 0xFFFF
            drng[b][pl.ds(k * 16, 16)] = lax.shift_right_logical(p, 16)

    def _fire(b):
        for q in range(GRP):
            pltpu.async_copy(tab_hbm.at[srng[b].at[pl.ds(q * CH, CH)]],
                             rows[b].at[pl.ds(q * CH, CH), :], gsem[b])

    def _wait(b):
        pltpu.make_async_copy(tab_hbm.at[srng[b]], rows[b], gsem[b]).wait()

    for b in range(NBUF):
        _unpack(b, b)
        _fire(b)

    zero16 = jnp.zeros((16,), jnp.float32)

    def _zrow(r, _):
        def _zcol(k, _):
            zbuf[r, pl.ds(k * 16, 16)] = zero16
            return 0
        return lax.fori_loop(0, D // 16, _zcol, 0)

    lax.fori_loop(0, ZB, _zrow, 0)

    for j in range(RPT // ZB):
        pltpu.sync_copy(zbuf, acc.at[pl.ds(s * RPT + j * ZB, ZB), :])

    @pl.when(s == 0)
    def _zero_tail():
        pltpu.sync_copy(zbuf.at[pl.ds(0, TAIL), :],
                        acc.at[pl.ds(NS * RPT, TAIL), :])

    plsc.subcore_barrier()

    def _group(t, _):
        j0 = t * NBUF
        for b in range(NBUF):
            j = j0 + b
            _wait(b)
            pltpu.sync_copy(rows[b], acc.at[drng[b]], add=True)
            nxt = j + NBUF

            @pl.when(nxt < NCHW)
            def _refire():
                _unpack(nxt, b)
                _fire(b)
        return 0

    lax.fori_loop(0, NCHW // NBUF, _group, 0)
    for j in range((NCHW // NBUF) * NBUF, NCHW):
        b = j % NBUF
        _wait(b)
        pltpu.sync_copy(rows[b], acc.at[drng[b]], add=True)
    plsc.subcore_barrier()

    for j in range(RPT // ZR):
        r0 = s * RPT + j * ZR
        pltpu.sync_copy(acc.at[pl.ds(r0, ZR), :], out_hbm.at[c, pl.ds(r0, ZR), :])

    @pl.when(s == 0)
    def _write_tail():
        pltpu.sync_copy(acc.at[pl.ds(NS * RPT, TAIL), :],
                        out_hbm.at[c, pl.ds(NS * RPT, TAIL), :])


def _sc_segment_sum(table, packed):
    mesh = plsc.VectorSubcoreMesh(core_axis_name="c", subcore_axis_name="s")
    f = pl.kernel(
        _sc_segment_sum_body,
        out_type=jax.ShapeDtypeStruct((NC, N, D), jnp.float32),
        mesh=mesh,
        scratch_types=(
            [pltpu.VMEM((EPW,), jnp.int32)]
            + [pltpu.VMEM((CHB,), jnp.int32)] * (2 * NBUF)
            + [pltpu.VMEM((CHB, D), jnp.float32)] * NBUF
            + [pltpu.VMEM((ZB, D), jnp.float32),
               pltpu.VMEM_SHARED((NA, D), jnp.float32)]
            + [pltpu.SemaphoreType.DMA] * NBUF
        ),
    )
    return f(table, packed)


def _tc_relu_body(p_ref, w_ref, b_ref, o_ref):
    a = p_ref[0] + p_ref[1]
    h = jnp.dot(a, w_ref[...], preferred_element_type=jnp.float32)
    o_ref[...] = jnp.maximum(h + b_ref[...], 0.0)


def _tc_sigmoid_body(p_ref, w_ref, b_ref, o_ref):
    a = p_ref[0] + p_ref[1]
    h = jnp.dot(a, w_ref[...], preferred_element_type=jnp.float32)
    o_ref[...] = jax.nn.sigmoid(h + b_ref[...])


def _tc_layer(partials, w, b, body):
    rb = 2000
    grid = (N // rb,)
    return pl.pallas_call(
        body,
        grid=grid,
        in_specs=[
            pl.BlockSpec((NC, rb, D), lambda i: (0, i, 0)),
            pl.BlockSpec((D, D), lambda i: (0, 0)),
            pl.BlockSpec((1, D), lambda i: (0, 0)),
        ],
        out_specs=pl.BlockSpec((rb, D), lambda i: (i, 0)),
        out_shape=jax.ShapeDtypeStruct((N, D), jnp.float32),
    )(partials, w, b.reshape(1, D))


def kernel(x, edge_index, W1, b1, W2, b2):
    pad = EPAD - E
    src = jnp.concatenate([edge_index[0], jnp.zeros((pad,), jnp.int32)])
    dst = jnp.concatenate([edge_index[1], jnp.full((pad,), JUNK, jnp.int32)])
    packed = src + dst * 65536
    p1 = _sc_segment_sum(x, packed)
    h = _tc_layer(p1, W1, b1, _tc_relu_body)
    p2 = _sc_segment_sum(h, packed)
    return _tc_layer(p2, W2, b2, _tc_sigmoid_body)


# speedup vs baseline: 1.4193x; 1.0001x over previous
import jax
import jax.numpy as jnp
from jax import lax
from jax.experimental import pallas as pl
from jax.experimental.pallas import tpu as pltpu
import jax.experimental.pallas.tpu_sc as plsc

N = 10000
E = 320000
D = 128

NC = 2
NS = 16
NW = NC * NS

CH = 16
GRP = 1
CHB = CH * GRP
NCHW = -(-E // (NW * CHB))
EPW = NCHW * CHB
EPAD = NW * EPW
NBUF = 16
JUNK = N
NA = N + 8
RPT = 624
ZR = 208
ZB = 16
TAIL = N - NS * RPT


def _sc_segment_sum_body(tab_hbm, packed_hbm, out_hbm, *scr):
    c = lax.axis_index("c")
    s = lax.axis_index("s")
    wid = c * NS + s
    packed_v = scr[0]
    srng = scr[1:1 + NBUF]
    drng = scr[1 + NBUF:1 + 2 * NBUF]
    rows = scr[1 + 2 * NBUF:1 + 3 * NBUF]
    zbuf = scr[1 + 3 * NBUF]
    acc = scr[2 + 3 * NBUF]
    gsem = scr[3 + 3 * NBUF:3 + 4 * NBUF]

    pltpu.sync_copy(packed_hbm.at[pl.ds(wid * EPW, EPW)], packed_v)

    def _unpack(j, b):
        for k in range(CHB // 16):
            p = packed_v[pl.ds(j * CHB + k * 16, 16)]
            srng[b][pl.ds(k * 16, 16)] = p & 0xFFFF
            drng[b][pl.ds(k * 16, 16)] = lax.shift_right_logical(p, 16)

    def _fire(b):
        for q in range(GRP):
            pltpu.async_copy(tab_hbm.at[srng[b].at[pl.ds(q * CH, CH)]],
                             rows[b].at[pl.ds(q * CH, CH), :], gsem[b])

    def _wait(b):
        pltpu.make_async_copy(tab_hbm.at[srng[b]], rows[b], gsem[b]).wait()

    for b in range(NBUF):
        _unpack(b, b)
        _fire(b)

    zero16 = jnp.zeros((16,), jnp.float32)

    def _zrow(r, _):
        def _zcol(k, _):
            zbuf[r, pl.ds(k * 16, 16)] = zero16
            return 0
        return lax.fori_loop(0, D // 16, _zcol, 0)

    lax.fori_loop(0, ZB, _zrow, 0)

    for j in range(RPT // ZB):
        pltpu.sync_copy(zbuf, acc.at[pl.ds(s * RPT + j * ZB, ZB), :])

    @pl.when(s == 0)
    def _zero_tail():
        pltpu.sync_copy(zbuf.at[pl.ds(0, TAIL), :],
                        acc.at[pl.ds(NS * RPT, TAIL), :])

    plsc.subcore_barrier()

    def _group(t, _):
        j0 = t * NBUF
        for b in range(NBUF):
            j = j0 + b
            _wait(b)
            pltpu.sync_copy(rows[b], acc.at[drng[b]], add=True)
            nxt = j + NBUF

            @pl.when(nxt < NCHW)
            def _refire():
                _unpack(nxt, b)
                _fire(b)
        return 0

    lax.fori_loop(0, NCHW // NBUF, _group, 0)
    for j in range((NCHW // NBUF) * NBUF, NCHW):
        b = j % NBUF
        _wait(b)
        pltpu.sync_copy(rows[b], acc.at[drng[b]], add=True)
    plsc.subcore_barrier()

    for j in range(RPT // ZR):
        r0 = s * RPT + j * ZR
        pltpu.sync_copy(acc.at[pl.ds(r0, ZR), :], out_hbm.at[c, pl.ds(r0, ZR), :])

    @pl.when(s == 0)
    def _write_tail():
        pltpu.sync_copy(acc.at[pl.ds(NS * RPT, TAIL), :],
                        out_hbm.at[c, pl.ds(NS * RPT, TAIL), :])


def _sc_segment_sum(table, packed):
    mesh = plsc.VectorSubcoreMesh(core_axis_name="c", subcore_axis_name="s")
    f = pl.kernel(
        _sc_segment_sum_body,
        out_type=jax.ShapeDtypeStruct((NC, N, D), jnp.float32),
        mesh=mesh,
        scratch_types=(
            [pltpu.VMEM((EPW,), jnp.int32)]
            + [pltpu.VMEM((CHB,), jnp.int32)] * (2 * NBUF)
            + [pltpu.VMEM((CHB, D), jnp.float32)] * NBUF
            + [pltpu.VMEM((ZB, D), jnp.float32),
               pltpu.VMEM_SHARED((NA, D), jnp.float32)]
            + [pltpu.SemaphoreType.DMA] * NBUF
        ),
    )
    return f(table, packed)


def _tc_relu_body(p_ref, w_ref, b_ref, o_ref):
    a = p_ref[0] + p_ref[1]
    h = jnp.dot(a, w_ref[...], preferred_element_type=jnp.float32)
    o_ref[...] = jnp.maximum(h + b_ref[...], 0.0)


def _tc_sigmoid_body(p_ref, w_ref, b_ref, o_ref):
    a = p_ref[0] + p_ref[1]
    h = jnp.dot(a, w_ref[...], preferred_element_type=jnp.float32)
    o_ref[...] = jax.nn.sigmoid(h + b_ref[...])


def _tc_layer(partials, w, b, body):
    rb = 2000
    grid = (N // rb,)
    return pl.pallas_call(
        body,
        grid=grid,
        in_specs=[
            pl.BlockSpec((NC, rb, D), lambda i: (0, i, 0)),
            pl.BlockSpec((D, D), lambda i: (0, 0)),
            pl.BlockSpec((1, D), lambda i: (0, 0)),
        ],
        out_specs=pl.BlockSpec((rb, D), lambda i: (i, 0)),
        out_shape=jax.ShapeDtypeStruct((N, D), jnp.float32),
    )(partials, w, b.reshape(1, D))


def kernel(x, edge_index, W1, b1, W2, b2):
    pad = EPAD - E
    if pad:
        src = jnp.concatenate([edge_index[0], jnp.zeros((pad,), jnp.int32)])
        dst = jnp.concatenate([edge_index[1], jnp.full((pad,), JUNK, jnp.int32)])
    else:
        src, dst = edge_index[0], edge_index[1]
    packed = src + dst * 65536
    p1 = _sc_segment_sum(x, packed)
    h = _tc_layer(p1, W1, b1, _tc_relu_body)
    p2 = _sc_segment_sum(h, packed)
    return _tc_layer(p2, W2, b2, _tc_sigmoid_body)
